# Initial kernel scaffold; baseline (speedup 1.0000x reference)
import jax
import jax.numpy as jnp
from jax.experimental import pallas as pl


def kernel(ref_feats, src_feats, ref_equ, src_equ, ref_masks, src_masks):
    raise NotImplementedError("write your pallas kernel here")


# speedup vs baseline: 138.9651x; 138.9651x over previous
import jax
import jax.numpy as jnp
from jax.experimental import pallas as pl

_NUM_CORRESPONDENCES = 2048
_GAMMA = 0.5
_N = 10000
_BM = 80
_V = 8


def _extract_body(ri_ref, cj_ref, s_ref, vals_ref, idxs_ref):
    blk = pl.program_id(0)
    sv = s_ref[...]
    riv = ri_ref[...].reshape(_BM, 1)
    cjv = cj_ref[...].reshape(1, _N)
    ref_ms = sv / riv
    src_ms = sv / cjv
    work = ref_ms * src_ms
    iota_j = jax.lax.broadcasted_iota(jnp.int32, (_BM, _N), 1)
    row = jax.lax.broadcasted_iota(jnp.int32, (_BM, 1), 0) + blk * _BM
    vcols = []
    icols = []
    for _ in range(_V):
        mx = jnp.max(work, axis=1, keepdims=True)
        sel = work == mx
        jsel = jnp.min(jnp.where(sel, iota_j, jnp.int32(2**30)), axis=1,
                       keepdims=True)
        vcols.append(mx)
        icols.append(row * _N + jsel)
        work = jnp.where(iota_j == jsel, -jnp.inf, work)
    vals_ref[...] = jnp.concatenate(vcols, axis=1)
    idxs_ref[...] = jnp.concatenate(icols, axis=1)


def _extract_candidates(s, ri, cj):
    grid = (_N // _BM,)
    return pl.pallas_call(
        _extract_body,
        grid=grid,
        in_specs=[
            pl.BlockSpec((1, 1, _BM), lambda i: (i, 0, 0)),
            pl.BlockSpec((1, _N), lambda i: (0, 0)),
            pl.BlockSpec((_BM, _N), lambda i: (i, 0)),
        ],
        out_specs=[
            pl.BlockSpec((_BM, _V), lambda i: (i, 0)),
            pl.BlockSpec((_BM, _V), lambda i: (i, 0)),
        ],
        out_shape=[
            jax.ShapeDtypeStruct((_N, _V), jnp.float32),
            jax.ShapeDtypeStruct((_N, _V), jnp.int32),
        ],
    )(ri.reshape(_N // _BM, 1, _BM), cj.reshape(1, _N), s)


def kernel(ref_feats, src_feats, ref_equ, src_equ, ref_masks, src_masks):
    ref_indices = jnp.nonzero(ref_masks, size=ref_masks.shape[0])[0]
    src_indices = jnp.nonzero(src_masks, size=src_masks.shape[0])[0]
    ref_feats = ref_feats[ref_indices]
    src_feats = src_feats[src_indices]
    ref_equ = ref_equ[ref_indices]
    src_equ = src_equ[src_indices]

    ms1 = jnp.clip(2.0 - 2.0 * jnp.matmul(ref_feats, src_feats.T), 0.0, None)
    a2 = jnp.sum(ref_equ * ref_equ, axis=-1)[:, None]
    b2 = jnp.sum(src_equ * src_equ, axis=-1)[None, :]
    ms2 = jnp.clip(a2 - 2.0 * jnp.matmul(ref_equ, src_equ.T) + b2, 0.0, None) * 0.1
    ms = (1.0 - _GAMMA) * ms1 + _GAMMA * ms2
    s = jnp.exp(-ms)
    ri = jnp.sum(s, axis=1, keepdims=True)
    cj = jnp.sum(s, axis=0, keepdims=True)

    vals, idxs = _extract_candidates(s, ri.reshape(_N), cj)

    kv = -jax.lax.bitcast_convert_type(vals.reshape(-1), jnp.int32)
    ki = idxs.reshape(-1)
    _, corr_indices, corr_scores = jax.lax.sort(
        (kv, ki, vals.reshape(-1)), num_keys=2)
    corr_indices = corr_indices[:_NUM_CORRESPONDENCES]
    corr_scores = corr_scores[:_NUM_CORRESPONDENCES]

    ref_sel_indices = corr_indices // _N
    src_sel_indices = corr_indices % _N
    ref_corr_indices = ref_indices[ref_sel_indices]
    src_corr_indices = src_indices[src_sel_indices]
    return (ref_corr_indices, src_corr_indices, corr_scores)
